# baseline (speedup 1.0000x reference)
import functools

import jax
import jax.numpy as jnp
from jax import lax
from jax.experimental import pallas as pl
from jax.experimental.pallas import tpu as pltpu
from jax.experimental.pallas import tpu_sc as plsc

N = 10000
E = 320000
H = 128

NC = 2
NS = 16
NW = NC * NS
CHG = 80
G = 5
EPW = E // NW
STEPS_G = EPW // CHG
KS = 80
GS = 2
CHS = KS * GS
ET = E // NS
STEPS_S = ET // CHS
NH = N // NC
NACC = 5120
RPT = NACC // NS
TRASH = NH

BN = 2000
BE = 16000

_f32 = jnp.float32


def _dot(a, b):
    return jnp.dot(a, b, preferred_element_type=_f32)


def _rows_spec(bs, w):
    return pl.BlockSpec((bs, w), lambda i: (i, 0))


def _full_spec(r, c):
    return pl.BlockSpec((r, c), lambda i: (0, 0))


def _node_encode_body(x_ref, mean_ref, std_ref, w1_ref, b1_ref, w2_ref, b2_ref,
                      ws_ref, bs_ref, wd_ref, x0_ref, ps_ref, pd_ref):
    xn = (x_ref[...] - mean_ref[...]) / std_ref[...]
    h = jnp.maximum(_dot(xn, w1_ref[...]) + b1_ref[...], 0.0)
    x0 = _dot(h, w2_ref[...]) + b2_ref[...]
    x0_ref[...] = x0
    ps_ref[...] = _dot(x0, ws_ref[...]) + bs_ref[...]
    pd_ref[...] = _dot(x0, wd_ref[...])


def _node_encode(x, mean, std, w1, b1, w2, b2, ws, bs, wd):
    return pl.pallas_call(
        _node_encode_body,
        grid=(N // BN,),
        in_specs=[_rows_spec(BN, H), _full_spec(1, H), _full_spec(1, H),
                  _full_spec(H, H), _full_spec(1, H), _full_spec(H, H),
                  _full_spec(1, H), _full_spec(H, H), _full_spec(1, H),
                  _full_spec(H, H)],
        out_specs=[_rows_spec(BN, H)] * 3,
        out_shape=[jax.ShapeDtypeStruct((N, H), _f32)] * 3,
    )(x, mean, std, w1, b1, w2, b2, ws, bs, wd)


def _edge_encode_body(e_ref, mean_ref, std_ref, w1_ref, b1_ref, w2_ref, b2_ref,
                      out_ref):
    en = (e_ref[...] - mean_ref[...]) / std_ref[...]
    h = jnp.maximum(_dot(en, w1_ref[...]) + b1_ref[...], 0.0)
    out_ref[...] = _dot(h, w2_ref[...]) + b2_ref[...]


def _edge_encode(e, mean, std, w1, b1, w2, b2):
    din = e.shape[1]
    return pl.pallas_call(
        _edge_encode_body,
        grid=(E // BE,),
        in_specs=[_rows_spec(BE, din), _full_spec(1, din), _full_spec(1, din),
                  _full_spec(din, H), _full_spec(1, H), _full_spec(H, H),
                  _full_spec(1, H)],
        out_specs=_rows_spec(BE, H),
        out_shape=jax.ShapeDtypeStruct((E, H), _f32),
    )(e, mean, std, w1, b1, w2, b2)


def _edge_enc_update_body(ea_ref, g_ref, mean_ref, std_ref, ew1_ref, eb1_ref,
                          ew2_ref, eb2_ref, we_ref, w2_ref, b2_ref, out_ref):
    en = (ea_ref[...] - mean_ref[...]) / std_ref[...]
    eh = jnp.maximum(_dot(en, ew1_ref[...]) + eb1_ref[...], 0.0)
    e0 = _dot(eh, ew2_ref[...]) + eb2_ref[...]
    h = jnp.maximum(_dot(e0, we_ref[...]) + g_ref[...], 0.0)
    out_ref[...] = _dot(h, w2_ref[...]) + b2_ref[...]


def _edge_enc_update(ea, g, mean, std, ew1, eb1, ew2, eb2, we, w2, b2):
    din = ea.shape[1]
    ne = ea.shape[0]
    return pl.pallas_call(
        _edge_enc_update_body,
        grid=(ne // BE,),
        in_specs=[_rows_spec(BE, din), _rows_spec(BE, H), _full_spec(1, din),
                  _full_spec(1, din), _full_spec(din, H), _full_spec(1, H),
                  _full_spec(H, H), _full_spec(1, H), _full_spec(H, H),
                  _full_spec(H, H), _full_spec(1, H)],
        out_specs=_rows_spec(BE, H),
        out_shape=jax.ShapeDtypeStruct((ne, H), _f32),
    )(ea, g, mean, std, ew1, eb1, ew2, eb2, we, w2, b2)


def _edge_update_body(e_ref, g_ref, we_ref, w2_ref, b2_ref, out_ref):
    h = jnp.maximum(_dot(e_ref[...], we_ref[...]) + g_ref[...], 0.0)
    out_ref[...] = _dot(h, w2_ref[...]) + b2_ref[...]


def _edge_update(e, g, we, w2, b2):
    ne = e.shape[0]
    return pl.pallas_call(
        _edge_update_body,
        grid=(ne // BE,),
        in_specs=[_rows_spec(BE, H), _rows_spec(BE, H),
                  _full_spec(H, H), _full_spec(H, H), _full_spec(1, H)],
        out_specs=_rows_spec(BE, H),
        out_shape=jax.ShapeDtypeStruct((ne, H), _f32),
    )(e, g, we, w2, b2)


def _node_update_body(x_ref, aga_ref, agb_ref, a1_ref, a2_ref, b1_ref, w2_ref,
                      b2_ref, ws_ref, bs_ref, wd_ref, x1_ref, ps_ref, pd_ref):
    agg = aga_ref[...] + agb_ref[...]
    h = jnp.maximum(_dot(x_ref[...], a1_ref[...]) + _dot(agg, a2_ref[...])
                    + b1_ref[...], 0.0)
    x1 = _dot(h, w2_ref[...]) + b2_ref[...]
    x1_ref[...] = x1
    ps_ref[...] = _dot(x1, ws_ref[...]) + bs_ref[...]
    pd_ref[...] = _dot(x1, wd_ref[...])


def _node_update(x, aga, agb, a1, a2, b1, w2, b2, ws, bs, wd):
    return pl.pallas_call(
        _node_update_body,
        grid=(N // BN,),
        in_specs=[_rows_spec(BN, H)] * 3 + [
            _full_spec(H, H), _full_spec(H, H), _full_spec(1, H),
            _full_spec(H, H), _full_spec(1, H), _full_spec(H, H),
            _full_spec(1, H), _full_spec(H, H)],
        out_specs=[_rows_spec(BN, H)] * 3,
        out_shape=[jax.ShapeDtypeStruct((N, H), _f32)] * 3,
    )(x, aga, agb, a1, a2, b1, w2, b2, ws, bs, wd)


def _node_final_body(x_ref, aga_ref, agb_ref, x0_ref, a1_ref, a2_ref, b1_ref,
                     w2_ref, b2_ref, dw1_ref, db1_ref, dw2_ref, db2_ref,
                     std_ref, mean_ref, out_ref):
    agg = aga_ref[...] + agb_ref[...]
    h = jnp.maximum(_dot(x_ref[...], a1_ref[...]) + _dot(agg, a2_ref[...])
                    + b1_ref[...], 0.0)
    x2 = _dot(h, w2_ref[...]) + b2_ref[...]
    xf = jnp.tanh(x2 + x0_ref[...])
    hh = jnp.maximum(_dot(xf, dw1_ref[...]) + db1_ref[...], 0.0)
    o = _dot(hh, dw2_ref[...]) + db2_ref[...]
    out_ref[...] = o * std_ref[...] + mean_ref[...]


def _node_final(x, aga, agb, x0, a1, a2, b1, w2, b2, dw1, db1, dw2p, db2p,
                stdp, meanp):
    return pl.pallas_call(
        _node_final_body,
        grid=(N // BN,),
        in_specs=[_rows_spec(BN, H)] * 4 + [
            _full_spec(H, H), _full_spec(H, H), _full_spec(1, H),
            _full_spec(H, H), _full_spec(1, H), _full_spec(H, H),
            _full_spec(1, H), _full_spec(H, H), _full_spec(1, H),
            _full_spec(1, H), _full_spec(1, H)],
        out_specs=_rows_spec(BN, H),
        out_shape=jax.ShapeDtypeStruct((N, H), _f32),
    )(x, aga, agb, x0, a1, a2, b1, w2, b2, dw1, db1, dw2p, db2p, stdp, meanp)


@functools.cache
def _build_gather_sum(ne, chg):
    epw = ne // NW
    steps_g = epw // chg
    mesh = plsc.VectorSubcoreMesh(core_axis_name="c", subcore_axis_name="s")

    @functools.partial(
        pl.kernel,
        mesh=mesh,
        out_type=jax.ShapeDtypeStruct((ne, H), _f32),
        scratch_types=[pltpu.VMEM((steps_g, 1, chg), jnp.int32),
                       pltpu.VMEM((steps_g, 1, chg), jnp.int32),
                       pltpu.VMEM((chg, H), _f32),
                       pltpu.VMEM((chg, H), _f32),
                       pltpu.VMEM((chg, H), _f32),
                       pltpu.SemaphoreType.DMA,
                       pltpu.SemaphoreType.DMA,
                       pltpu.SemaphoreType.DMA,
                       pltpu.SemaphoreType.DMA,
                       pltpu.SemaphoreType.DMA],
    )
    def gather_sum(ps_hbm, pd_hbm, src_hbm, dst_hbm, g_hbm,
                   si_all, di_all, rs0, rs1, rd,
                   gsem0, gsem1, dsem, wsem0, wsem1):
        cid = lax.axis_index("c")
        sid = lax.axis_index("s")
        wid = cid * NS + sid
        pltpu.sync_copy(src_hbm.at[wid], si_all)
        pltpu.sync_copy(dst_hbm.at[wid], di_all)
        bufs = ((rs0, gsem0, wsem0), (rs1, gsem1, wsem1))

        def fire_ps(j, rs_b, gsem_b):
            pltpu.async_copy(ps_hbm.at[si_all.at[j, 0]], rs_b, gsem_b)

        def fire_pd(j):
            pltpu.async_copy(pd_hbm.at[di_all.at[j, 0]], rd, dsem)

        def accumulate(rs_b):
            def row(r, carry):
                for k in range(H // 16):
                    sl = pl.ds(k * 16, 16)
                    plsc.addupdate(rs_b.at[r, sl], rd[r, sl])
                return carry
            lax.fori_loop(0, chg, row, 0)

        def process(j, b, prefire):
            rs_b, gsem_b, wsem_b = bufs[b]
            rs_a, gsem_a, wsem_a = bufs[1 - b]

            if prefire:
                @pl.when(j >= 1)
                def _():
                    pltpu.make_async_copy(
                        rs_a, g_hbm.at[pl.ds(0, chg)], wsem_a).wait()

                fire_ps(j + 1, rs_a, gsem_a)

            pltpu.make_async_copy(ps_hbm.at[pl.ds(0, chg)], rs_b,
                                  gsem_b).wait()
            pltpu.make_async_copy(pd_hbm.at[pl.ds(0, chg)], rd,
                                  dsem).wait()
            accumulate(rs_b)

            if prefire:
                fire_pd(j + 1)

            pltpu.async_copy(
                rs_b, g_hbm.at[pl.ds(wid * epw + j * chg, chg)], wsem_b)

        fire_ps(0, rs0, gsem0)
        fire_pd(0)

        def body(jj, carry):
            for b in range(2):
                j = jj * 2 + b
                process(j, b, True)
            return carry

        lax.fori_loop(0, steps_g // 2, body, 0)
        process(steps_g - 1, (steps_g - 1) % 2, False)
        pltpu.make_async_copy(rs0, g_hbm.at[pl.ds(0, chg)], wsem0).wait()
        pltpu.make_async_copy(rs1, g_hbm.at[pl.ds(0, chg)], wsem1).wait()

    return gather_sum


def _gather_sum(ps, pd, src4, dst4):
    ne = src4.shape[0] * src4.shape[1] * src4.shape[3]
    return _build_gather_sum(ne, src4.shape[3])(ps, pd, src4, dst4)


@functools.cache
def _build_segment_sum2(ne, gs_):
    et = ne // NS
    chs = KS * gs_
    steps_s = et // chs
    mesh = plsc.VectorSubcoreMesh(core_axis_name="c", subcore_axis_name="s")

    @functools.partial(
        pl.kernel,
        mesh=mesh,
        out_type=jax.ShapeDtypeStruct((2 * NACC, H), _f32),
        scratch_types=[pltpu.VMEM((gs_, KS), jnp.int32),
                       pltpu.VMEM((gs_, KS), jnp.int32),
                       pltpu.VMEM((gs_, KS), jnp.int32),
                       pltpu.VMEM((gs_, KS), jnp.int32),
                       pltpu.VMEM((chs, H), _f32),
                       pltpu.VMEM((chs, H), _f32),
                       pltpu.VMEM_SHARED((NACC, H), _f32),
                       pltpu.SemaphoreType.DMA,
                       pltpu.SemaphoreType.DMA,
                       pltpu.SemaphoreType.DMA,
                       pltpu.SemaphoreType.DMA],
    )
    def segment_sum2(vals_hbm, dst_hbm, zeros_hbm, out_hbm,
                     di0, di1, ti0, ti1, r0, r1, acc_sh,
                     isem0, isem1, vsem0, vsem1):
        cid = lax.axis_index("c")
        sid = lax.axis_index("s")
        base = cid * NH
        pltpu.sync_copy(zeros_hbm, acc_sh.at[pl.ds(sid * RPT, RPT)])
        plsc.subcore_barrier()
        bufs = ((r0, di0, ti0, isem0, vsem0), (r1, di1, ti1, isem1, vsem1))

        def fire(j, r_b, di_b, isem_b, vsem_b):
            pltpu.async_copy(dst_hbm.at[sid, j], di_b, isem_b)
            pltpu.async_copy(vals_hbm.at[pl.ds(sid * et + j * chs, chs)],
                             r_b, vsem_b)

        def remap(di_b, ti_b):
            for g in range(gs_):
                for k in range(KS // 16):
                    sl = pl.ds(k * 16, 16)
                    v = di_b[g, sl]
                    rel = v - base
                    ok = (rel >= 0) & (rel < NH)
                    ti_b[g, sl] = jnp.where(ok, rel, TRASH + (v & 63))

        def process(j, b, prefire):
            r_b, di_b, ti_b, isem_b, vsem_b = bufs[b]
            r_a, di_a, ti_a, isem_a, vsem_a = bufs[1 - b]
            if prefire:
                fire(j + 1, r_a, di_a, isem_a, vsem_a)
            pltpu.make_async_copy(dst_hbm.at[sid, 0], di_b, isem_b).wait()
            remap(di_b, ti_b)
            pltpu.make_async_copy(vals_hbm.at[pl.ds(0, chs)], r_b,
                                  vsem_b).wait()
            for g in range(gs_):
                pltpu.sync_copy(r_b.at[pl.ds(g * KS, KS)],
                                acc_sh.at[ti_b.at[g]], add=True)

        fire(0, r0, di0, isem0, vsem0)

        def body(jj, carry):
            for b in range(2):
                j = jj * 2 + b
                process(j, b, True)
            return carry

        lax.fori_loop(0, steps_s // 2, body, 0)
        process(steps_s - 1, (steps_s - 1) % 2, False)
        plsc.subcore_barrier()
        pltpu.sync_copy(acc_sh.at[pl.ds(sid * RPT, RPT)],
                        out_hbm.at[pl.ds(cid * NACC + sid * RPT, RPT)])

    return segment_sum2


def _segment_sum2(vals, dst_s, zeros_tile):
    return _build_segment_sum2(vals.shape[0], dst_s.shape[2])(
        vals, dst_s, zeros_tile)


def kernel(x, edge_attr, edge_index, ne_w1, ne_b1, ne_w2, ne_b2, ee_w1, ee_b1,
           ee_w2, ee_b2, gn_em_w1, gn_em_b1, gn_em_w2, gn_em_b2, gn_nm_w1,
           gn_nm_b1, gn_nm_w2, gn_nm_b2, de_w1, de_b1, de_w2, de_b2,
           node_mean, node_std, edge_mean, edge_std, out_mean, out_std):
    r1 = lambda v: v.reshape(1, -1)
    eh = E // 2
    chg_h = 40
    src_a = edge_index[0, :eh].reshape(NW, eh // NW // chg_h, 1, chg_h)
    src_b = edge_index[0, eh:].reshape(NW, eh // NW // chg_h, 1, chg_h)
    dst_a = edge_index[1, :eh].reshape(NW, eh // NW // chg_h, 1, chg_h)
    dst_b = edge_index[1, eh:].reshape(NW, eh // NW // chg_h, 1, chg_h)
    dst_as = edge_index[1, :eh].reshape(NS, eh // NS // KS, 1, KS)
    dst_bs = edge_index[1, eh:].reshape(NS, eh // NS // KS, 1, KS)
    zeros_tile = jnp.zeros((RPT, H), _f32)

    we = [gn_em_w1[i][:H] for i in range(2)]
    ws = [gn_em_w1[i][H:2 * H] for i in range(2)]
    wd = [gn_em_w1[i][2 * H:] for i in range(2)]
    na1 = [gn_nm_w1[i][:H] for i in range(2)]
    na2 = [gn_nm_w1[i][H:] for i in range(2)]

    x0, ps, pd = _node_encode(x, r1(node_mean), r1(node_std), ne_w1, r1(ne_b1),
                              ne_w2, r1(ne_b2), ws[0], r1(gn_em_b1[0]), wd[0])

    def agg_of(parts):
        return jnp.concatenate([parts[:NH], parts[NACC:NACC + NH]], axis=0)

    g0a = _gather_sum(ps, pd, src_a, dst_a)
    g0b = _gather_sum(ps, pd, src_b, dst_b)
    e0a = _edge_enc_update(edge_attr[:eh], g0a, r1(edge_mean), r1(edge_std),
                           ee_w1, r1(ee_b1), ee_w2, r1(ee_b2), we[0],
                           gn_em_w2[0], r1(gn_em_b2[0]))
    e0b = _edge_enc_update(edge_attr[eh:], g0b, r1(edge_mean), r1(edge_std),
                           ee_w1, r1(ee_b1), ee_w2, r1(ee_b2), we[0],
                           gn_em_w2[0], r1(gn_em_b2[0]))
    p0a = _segment_sum2(e0a, dst_as, zeros_tile)
    p0b = _segment_sum2(e0b, dst_bs, zeros_tile)
    x1, ps1, pd1 = _node_update(x0, agg_of(p0a), agg_of(p0b), na1[0], na2[0],
                                r1(gn_nm_b1[0]), gn_nm_w2[0], r1(gn_nm_b2[0]),
                                ws[1], r1(gn_em_b1[1]), wd[1])

    g1a = _gather_sum(ps1, pd1, src_a, dst_a)
    g1b = _gather_sum(ps1, pd1, src_b, dst_b)
    e1a = _edge_update(e0a, g1a, we[1], gn_em_w2[1], r1(gn_em_b2[1]))
    e1b = _edge_update(e0b, g1b, we[1], gn_em_w2[1], r1(gn_em_b2[1]))
    p1a = _segment_sum2(e1a, dst_as, zeros_tile)
    p1b = _segment_sum2(e1b, dst_bs, zeros_tile)

    out_dim = de_w2.shape[1]
    dw2p = jnp.pad(de_w2, ((0, 0), (0, H - out_dim)))
    db2p = jnp.pad(de_b2, (0, H - out_dim))
    stdp = jnp.pad(out_std, (0, H - out_dim), constant_values=1.0)
    meanp = jnp.pad(out_mean, (0, H - out_dim))
    out_full = _node_final(x1, agg_of(p1a), agg_of(p1b), x0, na1[1], na2[1],
                           r1(gn_nm_b1[1]), gn_nm_w2[1], r1(gn_nm_b2[1]),
                           de_w1, r1(de_b1), dw2p, r1(db2p), r1(stdp),
                           r1(meanp))
    return out_full[:, :out_dim]


# speedup vs baseline: 1.2352x; 1.2352x over previous
import functools

import jax
import jax.numpy as jnp
from jax import lax
from jax.experimental import pallas as pl
from jax.experimental.pallas import tpu as pltpu
from jax.experimental.pallas import tpu_sc as plsc

N = 10000
E = 320000
H = 128

NC = 2
NS = 16
NW = NC * NS
CHG = 80
G = 5
EPW = E // NW
STEPS_G = EPW // CHG
NPAD = 10240
RPT = NPAD // NS

BN = 2000
BE = 16000

_f32 = jnp.float32


def _dot(a, b):
    return jnp.dot(a, b, preferred_element_type=_f32)


def _rows_spec(bs, w):
    return pl.BlockSpec((bs, w), lambda i: (i, 0))


def _full_spec(r, c):
    return pl.BlockSpec((r, c), lambda i: (0, 0))


def _node_encode_body(x_ref, mean_ref, std_ref, w1_ref, b1_ref, w2_ref, b2_ref,
                      ws_ref, bs_ref, wd_ref, x0_ref, ps_ref, pd_ref):
    xn = (x_ref[...] - mean_ref[...]) / std_ref[...]
    h = jnp.maximum(_dot(xn, w1_ref[...]) + b1_ref[...], 0.0)
    x0 = _dot(h, w2_ref[...]) + b2_ref[...]
    x0_ref[...] = x0
    ps_ref[...] = _dot(x0, ws_ref[...]) + bs_ref[...]
    pd_ref[...] = _dot(x0, wd_ref[...])


def _node_encode(x, mean, std, w1, b1, w2, b2, ws, bs, wd):
    return pl.pallas_call(
        _node_encode_body,
        grid=(N // BN,),
        in_specs=[_rows_spec(BN, H), _full_spec(1, H), _full_spec(1, H),
                  _full_spec(H, H), _full_spec(1, H), _full_spec(H, H),
                  _full_spec(1, H), _full_spec(H, H), _full_spec(1, H),
                  _full_spec(H, H)],
        out_specs=[_rows_spec(BN, H)] * 3,
        out_shape=[jax.ShapeDtypeStruct((N, H), _f32)] * 3,
    )(x, mean, std, w1, b1, w2, b2, ws, bs, wd)


def _edge_encode_body(e_ref, mean_ref, std_ref, w1_ref, b1_ref, w2_ref, b2_ref,
                      out_ref):
    en = (e_ref[...] - mean_ref[...]) / std_ref[...]
    h = jnp.maximum(_dot(en, w1_ref[...]) + b1_ref[...], 0.0)
    out_ref[...] = _dot(h, w2_ref[...]) + b2_ref[...]


def _edge_encode(e, mean, std, w1, b1, w2, b2):
    din = e.shape[1]
    return pl.pallas_call(
        _edge_encode_body,
        grid=(E // BE,),
        in_specs=[_rows_spec(BE, din), _full_spec(1, din), _full_spec(1, din),
                  _full_spec(din, H), _full_spec(1, H), _full_spec(H, H),
                  _full_spec(1, H)],
        out_specs=_rows_spec(BE, H),
        out_shape=jax.ShapeDtypeStruct((E, H), _f32),
    )(e, mean, std, w1, b1, w2, b2)


def _edge_enc_update_body(ea_ref, g_ref, mean_ref, std_ref, ew1_ref, eb1_ref,
                          ew2_ref, eb2_ref, we_ref, w2_ref, b2_ref, out_ref):
    en = (ea_ref[...] - mean_ref[...]) / std_ref[...]
    eh = jnp.maximum(_dot(en, ew1_ref[...]) + eb1_ref[...], 0.0)
    e0 = _dot(eh, ew2_ref[...]) + eb2_ref[...]
    h = jnp.maximum(_dot(e0, we_ref[...]) + g_ref[...], 0.0)
    out_ref[...] = _dot(h, w2_ref[...]) + b2_ref[...]


def _edge_enc_update(ea, g, mean, std, ew1, eb1, ew2, eb2, we, w2, b2):
    din = ea.shape[1]
    ne = ea.shape[0]
    return pl.pallas_call(
        _edge_enc_update_body,
        grid=(ne // BE,),
        in_specs=[_rows_spec(BE, din), _rows_spec(BE, H), _full_spec(1, din),
                  _full_spec(1, din), _full_spec(din, H), _full_spec(1, H),
                  _full_spec(H, H), _full_spec(1, H), _full_spec(H, H),
                  _full_spec(H, H), _full_spec(1, H)],
        out_specs=_rows_spec(BE, H),
        out_shape=jax.ShapeDtypeStruct((ne, H), _f32),
    )(ea, g, mean, std, ew1, eb1, ew2, eb2, we, w2, b2)


def _edge_update_body(e_ref, g_ref, we_ref, w2_ref, b2_ref, out_ref):
    h = jnp.maximum(_dot(e_ref[...], we_ref[...]) + g_ref[...], 0.0)
    out_ref[...] = _dot(h, w2_ref[...]) + b2_ref[...]


def _edge_update(e, g, we, w2, b2):
    ne = e.shape[0]
    return pl.pallas_call(
        _edge_update_body,
        grid=(ne // BE,),
        in_specs=[_rows_spec(BE, H), _rows_spec(BE, H),
                  _full_spec(H, H), _full_spec(H, H), _full_spec(1, H)],
        out_specs=_rows_spec(BE, H),
        out_shape=jax.ShapeDtypeStruct((ne, H), _f32),
    )(e, g, we, w2, b2)


def _node_update_body(x_ref, aga_ref, agb_ref, a1_ref, a2_ref, b1_ref, w2_ref,
                      b2_ref, ws_ref, bs_ref, wd_ref, x1_ref, ps_ref, pd_ref):
    agg = aga_ref[...] + agb_ref[...]
    h = jnp.maximum(_dot(x_ref[...], a1_ref[...]) + _dot(agg, a2_ref[...])
                    + b1_ref[...], 0.0)
    x1 = _dot(h, w2_ref[...]) + b2_ref[...]
    x1_ref[...] = x1
    ps_ref[...] = _dot(x1, ws_ref[...]) + bs_ref[...]
    pd_ref[...] = _dot(x1, wd_ref[...])


def _node_update(x, aga, agb, a1, a2, b1, w2, b2, ws, bs, wd):
    return pl.pallas_call(
        _node_update_body,
        grid=(N // BN,),
        in_specs=[_rows_spec(BN, H)] * 3 + [
            _full_spec(H, H), _full_spec(H, H), _full_spec(1, H),
            _full_spec(H, H), _full_spec(1, H), _full_spec(H, H),
            _full_spec(1, H), _full_spec(H, H)],
        out_specs=[_rows_spec(BN, H)] * 3,
        out_shape=[jax.ShapeDtypeStruct((N, H), _f32)] * 3,
    )(x, aga, agb, a1, a2, b1, w2, b2, ws, bs, wd)


def _node_final_body(x_ref, aga_ref, agb_ref, x0_ref, a1_ref, a2_ref, b1_ref,
                     w2_ref, b2_ref, dw1_ref, db1_ref, dw2_ref, db2_ref,
                     std_ref, mean_ref, out_ref):
    agg = aga_ref[...] + agb_ref[...]
    h = jnp.maximum(_dot(x_ref[...], a1_ref[...]) + _dot(agg, a2_ref[...])
                    + b1_ref[...], 0.0)
    x2 = _dot(h, w2_ref[...]) + b2_ref[...]
    xf = jnp.tanh(x2 + x0_ref[...])
    hh = jnp.maximum(_dot(xf, dw1_ref[...]) + db1_ref[...], 0.0)
    o = _dot(hh, dw2_ref[...]) + db2_ref[...]
    out_ref[...] = o * std_ref[...] + mean_ref[...]


def _node_final(x, aga, agb, x0, a1, a2, b1, w2, b2, dw1, db1, dw2p, db2p,
                stdp, meanp):
    return pl.pallas_call(
        _node_final_body,
        grid=(N // BN,),
        in_specs=[_rows_spec(BN, H)] * 4 + [
            _full_spec(H, H), _full_spec(H, H), _full_spec(1, H),
            _full_spec(H, H), _full_spec(1, H), _full_spec(H, H),
            _full_spec(1, H), _full_spec(H, H), _full_spec(1, H),
            _full_spec(1, H), _full_spec(1, H)],
        out_specs=_rows_spec(BN, H),
        out_shape=jax.ShapeDtypeStruct((N, H), _f32),
    )(x, aga, agb, x0, a1, a2, b1, w2, b2, dw1, db1, dw2p, db2p, stdp, meanp)


@functools.cache
def _build_gather_sum(ne, chg):
    epw = ne // NW
    steps_g = epw // chg
    mesh = plsc.VectorSubcoreMesh(core_axis_name="c", subcore_axis_name="s")

    @functools.partial(
        pl.kernel,
        mesh=mesh,
        out_type=jax.ShapeDtypeStruct((ne, H), _f32),
        scratch_types=[pltpu.VMEM((steps_g, 1, chg), jnp.int32),
                       pltpu.VMEM((steps_g, 1, chg), jnp.int32),
                       pltpu.VMEM((chg, H), _f32),
                       pltpu.VMEM((chg, H), _f32),
                       pltpu.VMEM((chg, H), _f32),
                       pltpu.SemaphoreType.DMA,
                       pltpu.SemaphoreType.DMA,
                       pltpu.SemaphoreType.DMA,
                       pltpu.SemaphoreType.DMA,
                       pltpu.SemaphoreType.DMA],
    )
    def gather_sum(ps_hbm, pd_hbm, src_hbm, dst_hbm, g_hbm,
                   si_all, di_all, rs0, rs1, rd,
                   gsem0, gsem1, dsem, wsem0, wsem1):
        cid = lax.axis_index("c")
        sid = lax.axis_index("s")
        wid = cid * NS + sid
        pltpu.sync_copy(src_hbm.at[wid], si_all)
        pltpu.sync_copy(dst_hbm.at[wid], di_all)
        bufs = ((rs0, gsem0, wsem0), (rs1, gsem1, wsem1))

        def fire_ps(j, rs_b, gsem_b):
            pltpu.async_copy(ps_hbm.at[si_all.at[j, 0]], rs_b, gsem_b)

        def fire_pd(j):
            pltpu.async_copy(pd_hbm.at[di_all.at[j, 0]], rd, dsem)

        def accumulate(rs_b):
            def row(r, carry):
                for k in range(H // 16):
                    sl = pl.ds(k * 16, 16)
                    plsc.addupdate(rs_b.at[r, sl], rd[r, sl])
                return carry
            lax.fori_loop(0, chg, row, 0)

        def process(j, b, prefire):
            rs_b, gsem_b, wsem_b = bufs[b]
            rs_a, gsem_a, wsem_a = bufs[1 - b]

            if prefire:
                @pl.when(j >= 1)
                def _():
                    pltpu.make_async_copy(
                        rs_a, g_hbm.at[pl.ds(0, chg)], wsem_a).wait()

                fire_ps(j + 1, rs_a, gsem_a)

            pltpu.make_async_copy(ps_hbm.at[pl.ds(0, chg)], rs_b,
                                  gsem_b).wait()
            pltpu.make_async_copy(pd_hbm.at[pl.ds(0, chg)], rd,
                                  dsem).wait()
            accumulate(rs_b)

            if prefire:
                fire_pd(j + 1)

            pltpu.async_copy(
                rs_b, g_hbm.at[pl.ds(wid * epw + j * chg, chg)], wsem_b)

        fire_ps(0, rs0, gsem0)
        fire_pd(0)

        def body(jj, carry):
            for b in range(2):
                j = jj * 2 + b
                process(j, b, True)
            return carry

        lax.fori_loop(0, steps_g // 2, body, 0)
        process(steps_g - 1, (steps_g - 1) % 2, False)
        pltpu.make_async_copy(rs0, g_hbm.at[pl.ds(0, chg)], wsem0).wait()
        pltpu.make_async_copy(rs1, g_hbm.at[pl.ds(0, chg)], wsem1).wait()

    return gather_sum


def _gather_sum(ps, pd, src4, dst4):
    ne = src4.shape[0] * src4.shape[1] * src4.shape[3]
    return _build_gather_sum(ne, src4.shape[3])(ps, pd, src4, dst4)


@functools.cache
def _build_segment_sum2():
    mesh = plsc.VectorSubcoreMesh(core_axis_name="c", subcore_axis_name="s")

    @functools.partial(
        pl.kernel,
        mesh=mesh,
        out_type=jax.ShapeDtypeStruct((2 * NPAD, H), _f32),
        scratch_types=[pltpu.VMEM((1, CHG), jnp.int32),
                       pltpu.VMEM((1, CHG), jnp.int32),
                       pltpu.VMEM((CHG, H), _f32),
                       pltpu.VMEM((CHG, H), _f32),
                       pltpu.VMEM_SHARED((NPAD, H), _f32),
                       pltpu.SemaphoreType.DMA,
                       pltpu.SemaphoreType.DMA,
                       pltpu.SemaphoreType.DMA,
                       pltpu.SemaphoreType.DMA],
    )
    def segment_sum2(vals_hbm, dst_hbm, zeros_hbm, out_hbm,
                     di0, di1, r0, r1, acc_sh, isem0, isem1, vsem0, vsem1):
        cid = lax.axis_index("c")
        sid = lax.axis_index("s")
        wid = cid * NS + sid
        pltpu.sync_copy(zeros_hbm, acc_sh.at[pl.ds(sid * RPT, RPT)])
        plsc.subcore_barrier()
        bufs = ((r0, di0, isem0, vsem0), (r1, di1, isem1, vsem1))
        steps = EPW // CHG

        def fire(j, r_b, di_b, isem_b, vsem_b):
            pltpu.async_copy(dst_hbm.at[wid, j], di_b, isem_b)
            pltpu.async_copy(vals_hbm.at[pl.ds(wid * EPW + j * CHG, CHG)],
                             r_b, vsem_b)

        def process(j, b, prefire):
            r_b, di_b, isem_b, vsem_b = bufs[b]
            r_a, di_a, isem_a, vsem_a = bufs[1 - b]
            if prefire:
                fire(j + 1, r_a, di_a, isem_a, vsem_a)
            pltpu.make_async_copy(dst_hbm.at[wid, 0], di_b, isem_b).wait()
            pltpu.make_async_copy(vals_hbm.at[pl.ds(0, CHG)], r_b,
                                  vsem_b).wait()
            pltpu.sync_copy(r_b, acc_sh.at[di_b.at[0]], add=True)

        fire(0, r0, di0, isem0, vsem0)

        def body(jj, carry):
            for b in range(2):
                j = jj * 2 + b
                process(j, b, True)
            return carry

        lax.fori_loop(0, steps // 2, body, 0)
        process(steps - 1, (steps - 1) % 2, False)
        plsc.subcore_barrier()
        pltpu.sync_copy(acc_sh.at[pl.ds(sid * RPT, RPT)],
                        out_hbm.at[pl.ds(cid * NPAD + sid * RPT, RPT)])

    return segment_sum2


def _segment_sum2(vals, dst4, zeros_tile):
    return _build_segment_sum2()(vals, dst4, zeros_tile)


def kernel(x, edge_attr, edge_index, ne_w1, ne_b1, ne_w2, ne_b2, ee_w1, ee_b1,
           ee_w2, ee_b2, gn_em_w1, gn_em_b1, gn_em_w2, gn_em_b2, gn_nm_w1,
           gn_nm_b1, gn_nm_w2, gn_nm_b2, de_w1, de_b1, de_w2, de_b2,
           node_mean, node_std, edge_mean, edge_std, out_mean, out_std):
    r1 = lambda v: v.reshape(1, -1)
    src4 = edge_index[0].reshape(NW, STEPS_G, 1, CHG)
    dst4 = edge_index[1].reshape(NW, STEPS_G, 1, CHG)
    zeros_tile = jnp.zeros((RPT, H), _f32)

    we = [gn_em_w1[i][:H] for i in range(2)]
    ws = [gn_em_w1[i][H:2 * H] for i in range(2)]
    wd = [gn_em_w1[i][2 * H:] for i in range(2)]
    na1 = [gn_nm_w1[i][:H] for i in range(2)]
    na2 = [gn_nm_w1[i][H:] for i in range(2)]

    x0, ps, pd = _node_encode(x, r1(node_mean), r1(node_std), ne_w1, r1(ne_b1),
                              ne_w2, r1(ne_b2), ws[0], r1(gn_em_b1[0]), wd[0])

    g0 = _gather_sum(ps, pd, src4, dst4)
    e0 = _edge_enc_update(edge_attr, g0, r1(edge_mean), r1(edge_std), ee_w1,
                          r1(ee_b1), ee_w2, r1(ee_b2), we[0], gn_em_w2[0],
                          r1(gn_em_b2[0]))
    p0 = _segment_sum2(e0, dst4, zeros_tile)
    x1, ps1, pd1 = _node_update(x0, p0[:N], p0[NPAD:NPAD + N], na1[0], na2[0],
                                r1(gn_nm_b1[0]), gn_nm_w2[0], r1(gn_nm_b2[0]),
                                ws[1], r1(gn_em_b1[1]), wd[1])

    g1 = _gather_sum(ps1, pd1, src4, dst4)
    e1 = _edge_update(e0, g1, we[1], gn_em_w2[1], r1(gn_em_b2[1]))
    p1 = _segment_sum2(e1, dst4, zeros_tile)

    out_dim = de_w2.shape[1]
    dw2p = jnp.pad(de_w2, ((0, 0), (0, H - out_dim)))
    db2p = jnp.pad(de_b2, (0, H - out_dim))
    stdp = jnp.pad(out_std, (0, H - out_dim), constant_values=1.0)
    meanp = jnp.pad(out_mean, (0, H - out_dim))
    out_full = _node_final(x1, p1[:N], p1[NPAD:NPAD + N], x0, na1[1], na2[1],
                           r1(gn_nm_b1[1]), gn_nm_w2[1], r1(gn_nm_b2[1]),
                           de_w1, r1(de_b1), dw2p, r1(db2p), r1(stdp),
                           r1(meanp))
    return out_full[:, :out_dim]
